# baseline (speedup 1.0000x reference)
import functools

import jax
import jax.numpy as jnp
from jax import lax
from jax.experimental import pallas as pl
from jax.experimental.pallas import tpu as pltpu
from jax.experimental.pallas import tpu_sc as plsc

EMBED_MAX_NORM = 1.0
VOCAB_TILE = 2048


def _make_sc_gather(B, V, D):
    info = plsc.get_sparse_core_info()
    NC, NS = info.num_cores, info.num_subcores
    NW = NC * NS
    assert B % (8 * NW) == 0
    b_per_w = B // NW
    mesh = plsc.VectorSubcoreMesh(core_axis_name="c", subcore_axis_name="s")

    @functools.partial(
        pl.kernel,
        mesh=mesh,
        out_type=jax.ShapeDtypeStruct((B, D), jnp.float32),
        scratch_types=[
            pltpu.VMEM((b_per_w,), jnp.int32),
            pltpu.VMEM((b_per_w, D), jnp.float32),
            pltpu.SemaphoreType.DMA,
        ],
        compiler_params=pltpu.CompilerParams(use_tc_tiling_on_sc=False),
    )
    def gather(idx_hbm, table_hbm, out_hbm, idx_v, rows_v, sem):
        wid = lax.axis_index("s") * NC + lax.axis_index("c")
        base = wid * b_per_w
        pltpu.sync_copy(idx_hbm.at[pl.ds(base, b_per_w)], idx_v)
        pltpu.async_copy(table_hbm.at[idx_v], rows_v, sem).wait()
        pltpu.sync_copy(rows_v, out_hbm.at[pl.ds(base, b_per_w)])

    return gather


def _prescale_body(x_ref, xs_ref):
    x = x_ref[...]
    norm = jnp.sqrt(jnp.sum(x * x, axis=1, keepdims=True))
    scale = jnp.where(
        norm > EMBED_MAX_NORM,
        EMBED_MAX_NORM / jnp.maximum(norm, 1e-12),
        1.0,
    )
    xs_ref[...] = x * scale


def _proj_body(x_ref, w_ref, b_ref, out_ref):
    acc = lax.dot_general(
        x_ref[...], w_ref[...],
        (((1,), (1,)), ((), ())),
        preferred_element_type=jnp.float32,
    )
    out_ref[...] = acc + b_ref[...]


def kernel(token, emb_table, W, b):
    B = token.shape[0]
    V, D = emb_table.shape

    x = _make_sc_gather(B, V, D)(token.astype(jnp.int32), emb_table)

    xs = pl.pallas_call(
        _prescale_body,
        out_shape=jax.ShapeDtypeStruct((B, D), jnp.float32),
    )(x)

    grid = (pl.cdiv(V, VOCAB_TILE),)
    logits = pl.pallas_call(
        _proj_body,
        grid=grid,
        in_specs=[
            pl.BlockSpec((B, D), lambda j: (0, 0)),
            pl.BlockSpec((VOCAB_TILE, D), lambda j: (j, 0)),
            pl.BlockSpec((1, VOCAB_TILE), lambda j: (0, j)),
        ],
        out_specs=pl.BlockSpec((B, VOCAB_TILE), lambda j: (0, j)),
        out_shape=jax.ShapeDtypeStruct((B, V), jnp.float32),
        compiler_params=pltpu.CompilerParams(
            dimension_semantics=("parallel",),
        ),
    )(xs, W, b.reshape(1, V))
    return logits


# speedup vs baseline: 1.0155x; 1.0155x over previous
import functools

import jax
import jax.numpy as jnp
from jax import lax
from jax.experimental import pallas as pl
from jax.experimental.pallas import tpu as pltpu
from jax.experimental.pallas import tpu_sc as plsc

EMBED_MAX_NORM = 1.0
VOCAB_TILE = 2048


def _make_sc_gather(B, V, D):
    info = plsc.get_sparse_core_info()
    NC, NS = info.num_cores, info.num_subcores
    NW = NC * NS
    assert B % (8 * NW) == 0
    b_per_w = B // NW
    mesh = plsc.VectorSubcoreMesh(core_axis_name="c", subcore_axis_name="s")

    @functools.partial(
        pl.kernel,
        mesh=mesh,
        out_type=jax.ShapeDtypeStruct((B, D), jnp.float32),
        scratch_types=[
            pltpu.VMEM((b_per_w,), jnp.int32),
            pltpu.VMEM((b_per_w, D), jnp.float32),
            pltpu.SemaphoreType.DMA,
        ],
        compiler_params=pltpu.CompilerParams(use_tc_tiling_on_sc=False),
    )
    def gather(idx_hbm, table_hbm, out_hbm, idx_v, rows_v, sem):
        wid = lax.axis_index("s") * NC + lax.axis_index("c")
        base = wid * b_per_w
        pltpu.sync_copy(idx_hbm.at[pl.ds(base, b_per_w)], idx_v)
        pltpu.async_copy(table_hbm.at[idx_v], rows_v, sem).wait()
        pltpu.sync_copy(rows_v, out_hbm.at[pl.ds(base, b_per_w)])

    return gather


def _prescale_body(x_ref, xs_ref):
    x = x_ref[...]
    norm = jnp.sqrt(jnp.sum(x * x, axis=1, keepdims=True))
    scale = jnp.where(
        norm > EMBED_MAX_NORM,
        EMBED_MAX_NORM / jnp.maximum(norm, 1e-12),
        1.0,
    )
    xs_ref[...] = x * scale


NBUF = 4


def _make_proj_body(V):
    VT = VOCAB_TILE
    nj = pl.cdiv(V, VT)
    rem = V - (nj - 1) * VT

    def body(x_ref, w_ref, b_ref, out_ref, buf_ref, tail_ref, sem):
        j = pl.program_id(0)
        slot = lax.rem(j, NBUF)

        @pl.when(j >= NBUF)
        def _():
            pltpu.make_async_copy(
                buf_ref.at[0],
                out_ref.at[:, pl.ds(0, VT)],
                sem.at[slot],
            ).wait()

        acc = lax.dot_general(
            x_ref[...], w_ref[...],
            (((1,), (1,)), ((), ())),
            preferred_element_type=jnp.float32,
        ) + b_ref[...]

        @pl.when(j < nj - 1)
        def _():
            buf_ref[slot] = acc
            pltpu.make_async_copy(
                buf_ref.at[slot],
                out_ref.at[:, pl.ds(j * VT, VT)],
                sem.at[slot],
            ).start()

        @pl.when(j == nj - 1)
        def _():
            tail_ref[...] = acc[:, :rem]
            pltpu.make_async_copy(
                tail_ref,
                out_ref.at[:, pl.ds((nj - 1) * VT, rem)],
                sem.at[slot],
            ).start()
            last_slot = (nj - 1) % NBUF
            for k in range(1, NBUF + 1):
                s = (last_slot + k) % NBUF
                if s == last_slot:
                    pltpu.make_async_copy(
                        tail_ref,
                        out_ref.at[:, pl.ds((nj - 1) * VT, rem)],
                        sem.at[s],
                    ).wait()
                else:
                    pltpu.make_async_copy(
                        buf_ref.at[0],
                        out_ref.at[:, pl.ds(0, VT)],
                        sem.at[s],
                    ).wait()

    return body


def kernel(token, emb_table, W, b):
    B = token.shape[0]
    V, D = emb_table.shape
    VT = VOCAB_TILE

    x = _make_sc_gather(B, V, D)(token.astype(jnp.int32), emb_table)

    xs = pl.pallas_call(
        _prescale_body,
        out_shape=jax.ShapeDtypeStruct((B, D), jnp.float32),
    )(x)

    grid = (pl.cdiv(V, VT),)
    logits = pl.pallas_call(
        _make_proj_body(V),
        grid=grid,
        in_specs=[
            pl.BlockSpec((B, D), lambda j: (0, 0)),
            pl.BlockSpec((VT, D), lambda j: (j, 0)),
            pl.BlockSpec((1, VT), lambda j: (0, j)),
        ],
        out_specs=pl.BlockSpec(memory_space=pl.ANY),
        out_shape=jax.ShapeDtypeStruct((B, V), jnp.float32),
        scratch_shapes=[
            pltpu.VMEM((NBUF, B, VT), jnp.float32),
            pltpu.VMEM((B, V - (pl.cdiv(V, VT) - 1) * VT), jnp.float32),
            pltpu.SemaphoreType.DMA((NBUF,)),
        ],
    )(xs, W, b.reshape(1, V))
    return logits
